# Initial kernel scaffold; baseline (speedup 1.0000x reference)
import jax
import jax.numpy as jnp
from jax.experimental import pallas as pl


def kernel(class_logits, box_regression, features, proposal_boxes):
    raise NotImplementedError("write your pallas kernel here")


# speedup vs baseline: 1.3389x; 1.3389x over previous
import math

import jax
import jax.numpy as jnp
from jax.experimental import pallas as pl

N = 20000
C = 81
CM1 = C - 1
K = 500
SCORE_THRESH = 0.05
NMS_THRESH = 0.5
DET_PER_IMG = 100
IMG_W = 1333.0
IMG_H = 800.0
WX, WY, WW, WH = 10.0, 10.0, 5.0, 5.0
BBOX_XFORM_CLIP = math.log(1000.0 / 16.0)


def _softmax_mask_kernel(logits_ref, out_ref):
    x = logits_ref[...]
    m = jnp.max(x, axis=1, keepdims=True)
    e = jnp.exp(x - m)
    s = jnp.sum(e, axis=1, keepdims=True)
    p = e / s
    out_ref[...] = jnp.where(p > SCORE_THRESH, p, -jnp.inf)


def _nms_kernel(s_ref, rel_ref, prop_ref, sout_ref, box_ref):
    s = s_ref[0]
    r = rel_ref[0]
    p = prop_ref[0]

    w = p[2:3] - p[0:1] + 1.0
    h = p[3:4] - p[1:2] + 1.0
    cx = p[0:1] + 0.5 * w
    cy = p[1:2] + 0.5 * h
    dx = r[0:1] / WX
    dy = r[1:2] / WY
    dw = jnp.minimum(r[2:3] / WW, BBOX_XFORM_CLIP)
    dh = jnp.minimum(r[3:4] / WH, BBOX_XFORM_CLIP)
    pcx = dx * w + cx
    pcy = dy * h + cy
    pw = jnp.exp(dw) * w
    ph = jnp.exp(dh) * h
    x1 = jnp.clip(pcx - 0.5 * pw, 0.0, IMG_W - 1.0)
    y1 = jnp.clip(pcy - 0.5 * ph, 0.0, IMG_H - 1.0)
    x2 = jnp.clip(pcx + 0.5 * pw - 1.0, 0.0, IMG_W - 1.0)
    y2 = jnp.clip(pcy + 0.5 * ph - 1.0, 0.0, IMG_H - 1.0)

    area = (x2 - x1) * (y2 - y1)
    x1t = jnp.reshape(x1, (K, 1))
    y1t = jnp.reshape(y1, (K, 1))
    x2t = jnp.reshape(x2, (K, 1))
    y2t = jnp.reshape(y2, (K, 1))
    areat = jnp.reshape(area, (K, 1))
    xx1 = jnp.maximum(x1t, x1)
    yy1 = jnp.maximum(y1t, y1)
    xx2 = jnp.minimum(x2t, x2)
    yy2 = jnp.minimum(y2t, y2)
    inter = jnp.maximum(xx2 - xx1, 0.0) * jnp.maximum(yy2 - yy1, 0.0)
    union = areat + area - inter
    iou = inter / jnp.maximum(union, 1e-9)

    rowi = jax.lax.broadcasted_iota(jnp.int32, (K, K), 0)
    coli = jax.lax.broadcasted_iota(jnp.int32, (K, K), 1)
    sup = jnp.where((iou > NMS_THRESH) & (rowi < coli), 1.0, 0.0)

    validf = jnp.where(s > 0.0, 1.0, 0.0)

    def cond(carry):
        return carry[1]

    def body(carry):
        keep, _ = carry
        suppressed = jax.lax.dot_general(
            keep, sup, (((1,), (0,)), ((), ())),
            preferred_element_type=jnp.float32)
        newk = validf * jnp.where(suppressed > 0.0, 0.0, 1.0)
        return (newk, jnp.any(newk != keep))

    keep, _ = jax.lax.while_loop(cond, body, (validf, jnp.bool_(True)))

    sout_ref[0] = jnp.where(keep > 0.0, s, -jnp.inf)
    box_ref[0] = jnp.concatenate([x1, y1, x2, y2], axis=0)


def kernel(class_logits, box_regression, features, proposal_boxes):
    masked = pl.pallas_call(
        _softmax_mask_kernel,
        out_shape=jax.ShapeDtypeStruct((N, C), jnp.float32),
    )(class_logits)

    cls_scores = masked.T[1:]
    top_s, idx = jax.lax.top_k(cls_scores, K)

    rel = box_regression.reshape(N, C, 4)
    cls_ids = jnp.arange(1, C)[:, None]
    rel_t = rel[idx, cls_ids].transpose(0, 2, 1)
    prop_t = proposal_boxes[idx].transpose(0, 2, 1)

    s_out, box_t = pl.pallas_call(
        _nms_kernel,
        grid=(CM1,),
        in_specs=[
            pl.BlockSpec((1, 1, K), lambda c: (c, 0, 0)),
            pl.BlockSpec((1, 4, K), lambda c: (c, 0, 0)),
            pl.BlockSpec((1, 4, K), lambda c: (c, 0, 0)),
        ],
        out_specs=[
            pl.BlockSpec((1, 1, K), lambda c: (c, 0, 0)),
            pl.BlockSpec((1, 4, K), lambda c: (c, 0, 0)),
        ],
        out_shape=[
            jax.ShapeDtypeStruct((CM1, 1, K), jnp.float32),
            jax.ShapeDtypeStruct((CM1, 4, K), jnp.float32),
        ],
    )(top_s[:, None, :], rel_t, prop_t)

    flat_s = s_out.reshape(-1)
    flat_b = box_t.transpose(0, 2, 1).reshape(-1, 4)
    flat_idx = idx.reshape(-1)
    labels = jnp.broadcast_to(jnp.arange(1, C)[:, None], (CM1, K)).reshape(-1)

    top_s2, top_i = jax.lax.top_k(flat_s, DET_PER_IMG)
    final_b = flat_b[top_i]
    final_l = labels[top_i].astype(jnp.float32)
    final_feat = features[flat_idx[top_i]]
    final_s = jnp.where(jnp.isfinite(top_s2), top_s2, 0.0)
    return jnp.concatenate(
        [final_b, final_s[:, None], final_l[:, None], final_feat], axis=1)
